# baseline (speedup 1.0000x reference)
import jax
import jax.numpy as jnp
from jax.experimental import pallas as pl
from jax.experimental.pallas import tpu as pltpu


def _proj_kernel(f_ref, w_ref, o_ref):
    o_ref[:, :] = jnp.dot(f_ref[:, :], w_ref[:, :],
                          preferred_element_type=jnp.float32)


def _layer1_kernel(adj_ref, s1_ref, b1_ref, w2_ref, o_ref):
    y = jnp.dot(adj_ref[:, :], s1_ref[:, :],
                preferred_element_type=jnp.float32)
    x = jnp.maximum(y + b1_ref[:, :], 0.0)
    o_ref[:, :] = jnp.dot(x, w2_ref[:, :],
                          preferred_element_type=jnp.float32)


def _layer2_kernel(adj_ref, s2_ref, b2_ref, o_ref):
    y = jnp.dot(adj_ref[:, :], s2_ref[:, :],
                preferred_element_type=jnp.float32)
    o_ref[:, :] = jnp.maximum(y + b2_ref[:, :], 0.0)


def kernel(features, adj, W1, b1, W2, b2):
    n, feat = features.shape
    h1 = W1.shape[1]
    h2 = W2.shape[1]

    blk = min(n, 400)
    nb = pl.cdiv(n, blk)

    s1 = pl.pallas_call(
        _proj_kernel,
        out_shape=jax.ShapeDtypeStruct((n, h1), jnp.float32),
    )(features, W1)

    b1r = b1.reshape(1, h1)
    b2r = b2.reshape(1, h2)

    s2 = pl.pallas_call(
        _layer1_kernel,
        grid=(nb,),
        in_specs=[
            pl.BlockSpec((blk, n), lambda i: (i, 0)),
            pl.BlockSpec((n, h1), lambda i: (0, 0)),
            pl.BlockSpec((1, h1), lambda i: (0, 0)),
            pl.BlockSpec((h1, h2), lambda i: (0, 0)),
        ],
        out_specs=pl.BlockSpec((blk, h2), lambda i: (i, 0)),
        out_shape=jax.ShapeDtypeStruct((n, h2), jnp.float32),
        compiler_params=pltpu.CompilerParams(
            dimension_semantics=("parallel",)),
    )(adj, s1, b1r, W2)

    emb = pl.pallas_call(
        _layer2_kernel,
        grid=(nb,),
        in_specs=[
            pl.BlockSpec((blk, n), lambda i: (i, 0)),
            pl.BlockSpec((n, h2), lambda i: (0, 0)),
            pl.BlockSpec((1, h2), lambda i: (0, 0)),
        ],
        out_specs=pl.BlockSpec((blk, h2), lambda i: (i, 0)),
        out_shape=jax.ShapeDtypeStruct((n, h2), jnp.float32),
        compiler_params=pltpu.CompilerParams(
            dimension_semantics=("parallel",)),
    )(adj, s2, b2r)

    return emb
